# baseline (speedup 1.0000x reference)
import functools

import jax
import jax.numpy as jnp
from jax import lax
from jax.experimental import pallas as pl
from jax.experimental.pallas import tpu as pltpu
from jax.experimental.pallas import tpu_sc as plsc

E = 160000
N = 10000
H = 128
EF = 16
NB = 6
NMOL = 100
MOLSZ = 100

NC = 2
NS = 16
NW = NC * NS
EPW = E // NW
CE = 40
NCHUNK = EPW // CE
RN = 40
NODE_CHUNKS = N // RN
RB = 800
F32 = jnp.float32


def _mesh():
    return plsc.VectorSubcoreMesh(
        core_axis_name="c", subcore_axis_name="s", num_cores=NC, num_subcores=NS
    )


def _wid():
    return lax.axis_index("s") * NC + lax.axis_index("c")


def _sigmoid16(x):
    return 1.0 / (1.0 + jnp.exp(-x))


CG = 128
NCG = EPW // CG
TAILG = EPW - NCG * CG


@functools.cache
def _sc_gather_rows():
    @functools.partial(
        pl.kernel,
        out_type=jax.ShapeDtypeStruct((E, H), F32),
        mesh=_mesh(),
        scratch_types=[
            pltpu.VMEM((EPW,), jnp.int32),
            pltpu.VMEM((3, CG, H), F32),
            pltpu.SemaphoreType.DMA,
            pltpu.SemaphoreType.DMA,
            pltpu.SemaphoreType.DMA,
            pltpu.SemaphoreType.DMA,
            pltpu.SemaphoreType.DMA,
            pltpu.SemaphoreType.DMA,
        ],
    )
    def k(table_hbm, idx_hbm, out_hbm, idx_all, rows, g0, g1, g2, o0, o1, o2):
        gsems = (g0, g1, g2)
        osems = (o0, o1, o2)
        w0 = _wid() * EPW
        pltpu.sync_copy(idx_hbm.at[pl.ds(w0, EPW)], idx_all)

        def gcp(ci, b, n):
            off = pl.multiple_of(ci * CG, 8)
            return pltpu.make_async_copy(
                table_hbm.at[idx_all.at[pl.ds(off, n)]],
                rows.at[b, pl.ds(0, n)], gsems[b])

        def ow_start(ci, b, n):
            off = pl.multiple_of(ci * CG, 8)
            pltpu.async_copy(rows.at[b, pl.ds(0, n)],
                             out_hbm.at[pl.ds(w0 + off, n)], osems[b])

        def ow_wait(b, n):
            pltpu.make_async_copy(rows.at[b, pl.ds(0, n)],
                                  out_hbm.at[pl.ds(0, n)], osems[b]).wait()

        gcp(0, 0, CG).start()

        def outer(i, carry):
            for b in range(3):
                ci = i * 3 + b
                gcp(ci, b, CG).wait()
                nb = (b + 1) % 3

                @pl.when(ci + 1 < NCG)
                def _():
                    @pl.when(ci >= 2)
                    def _():
                        ow_wait(nb, CG)

                    gcp(ci + 1, nb, CG).start()

                ow_start(ci, b, CG)
            return carry

        lax.fori_loop(0, NCG // 3, outer, 0)
        ow_wait(1, CG)
        ow_wait(2, CG)
        ow_wait(0, CG)

        gcp(NCG, 0, TAILG).start()
        gcp(NCG, 0, TAILG).wait()
        ow_start(NCG, 0, TAILG)
        ow_wait(0, TAILG)

    return k


CE2 = 24
NCH2 = EPW // CE2
TAIL2 = EPW - NCH2 * CE2


@functools.cache
def _sc_msg():
    @functools.partial(
        pl.kernel,
        out_type=jax.ShapeDtypeStruct((E, 2 * H), F32),
        mesh=_mesh(),
        scratch_types=[
            pltpu.VMEM((NB * EPW,), jnp.int32),
            pltpu.VMEM((2, NB, CE2, 2 * H), F32),
            pltpu.VMEM((2, CE2, H), F32),
            pltpu.VMEM((2, CE2, 2 * H), F32),
            pltpu.SemaphoreType.DMA,
            pltpu.SemaphoreType.DMA,
            pltpu.SemaphoreType.DMA,
            pltpu.SemaphoreType.DMA,
        ],
    )
    def k(tab_hbm, rm_hbm, bgT_hbm, out_hbm, idx_all, gb, rmv, ob,
          gs0, gs1, os0, os1):
        gsems = (gs0, gs1)
        osems = (os0, os1)
        w0 = _wid() * EPW
        for j in range(NB):
            pltpu.sync_copy(bgT_hbm.at[pl.ds(j * E + w0, EPW)],
                            idx_all.at[pl.ds(j * EPW, EPW)])

        def gather_cps(ci, b, n):
            off = pl.multiple_of(ci * CE2, 8)
            cps = [
                pltpu.make_async_copy(
                    tab_hbm.at[idx_all.at[pl.ds(j * EPW + off, n)]],
                    gb.at[b, j, pl.ds(0, n)], gsems[b])
                for j in range(NB)
            ]
            cps.append(pltpu.make_async_copy(
                rm_hbm.at[pl.ds(w0 + off, n)],
                rmv.at[b, pl.ds(0, n)], gsems[b]))
            return cps

        def owrite_start(ci, b, n):
            off = pl.multiple_of(ci * CE2, 8)
            pltpu.async_copy(ob.at[b, pl.ds(0, n)],
                             out_hbm.at[pl.ds(w0 + off, n)], osems[b])

        def owrite_wait(b, n):
            pltpu.make_async_copy(ob.at[b, pl.ds(0, n)],
                                  out_hbm.at[pl.ds(0, n)], osems[b]).wait()

        def compute(b, n):
            def edge(e, ecarry):
                for sl in range(H // 16):
                    o = sl * 16
                    rv = rmv[b, e, pl.ds(o, 16)]
                    accs = jnp.zeros((16,), F32)
                    accg = jnp.zeros((16,), F32)
                    for j in range(NB):
                        hv = gb[b, j, e, pl.ds(o, 16)]
                        uv = gb[b, j, e, pl.ds(H + o, 16)]
                        g = 1.0 / (1.0 + jnp.exp(rv + uv))
                        accs = accs + hv
                        accg = accg + g * hv
                    ob[b, e, pl.ds(o, 16)] = accs
                    ob[b, e, pl.ds(H + o, 16)] = accg
                return ecarry

            lax.fori_loop(0, n, edge, 0, unroll=2)

        for cp in gather_cps(0, 0, CE2):
            cp.start()

        def outer(i, carry):
            for b in range(2):
                ci = i * 2 + b
                for cp in gather_cps(ci, b, CE2):
                    cp.wait()

                @pl.when(ci + 1 < NCH2)
                def _():
                    for cp in gather_cps(ci + 1, 1 - b, CE2):
                        cp.start()

                @pl.when(ci >= 2)
                def _():
                    owrite_wait(b, CE2)

                compute(b, CE2)
                owrite_start(ci, b, CE2)
            return carry

        lax.fori_loop(0, NCH2 // 2, outer, 0)
        owrite_wait(0, CE2)
        owrite_wait(1, CE2)

        for cp in gather_cps(NCH2, 0, TAIL2):
            cp.start()
        for cp in gather_cps(NCH2, 0, TAIL2):
            cp.wait()
        compute(0, TAIL2)
        owrite_start(NCH2, 0, TAIL2)
        owrite_wait(0, TAIL2)

    return k


@functools.cache
def _sc_nbr():
    @functools.partial(
        pl.kernel,
        out_type=jax.ShapeDtypeStruct((N, H), F32),
        mesh=_mesh(),
        scratch_types=[
            pltpu.VMEM((NB, RN), jnp.int32),
            pltpu.VMEM((NB, RN, H), F32),
            pltpu.VMEM((RN, H), F32),
            pltpu.SemaphoreType.DMA,
        ],
    )
    def k(h_hbm, agT_hbm, out_hbm, idx_v, gb_v, ob_v, sem):
        w = _wid()
        steps = (NODE_CHUNKS + NW - 1) // NW

        def step(si, carry):
            ci = w + si * NW

            @pl.when(ci < NODE_CHUNKS)
            def _():
                base = pl.multiple_of(ci * RN, 8)
                for j in range(NB):
                    pltpu.sync_copy(agT_hbm.at[pl.ds(j * N + base, RN)],
                                    idx_v.at[j])
                cps = [
                    pltpu.async_copy(h_hbm.at[idx_v.at[j]], gb_v.at[j], sem)
                    for j in range(NB)
                ]
                for cp in cps:
                    cp.wait()

                def node(e, ecarry):
                    for sl in range(H // 16):
                        o = sl * 16
                        acc = jnp.zeros((16,), F32)
                        for j in range(NB):
                            acc = acc + gb_v[j, e, pl.ds(o, 16)]
                        ob_v[e, pl.ds(o, 16)] = acc
                    return ecarry

                lax.fori_loop(0, RN, node, 0)
                pltpu.sync_copy(ob_v, out_hbm.at[pl.ds(base, RN)])

            return carry

        lax.fori_loop(0, steps, step, 0)

    return k


def _dot(a, b):
    return jnp.dot(a, b, preferred_element_type=F32)


def _mask_row0(x):
    rows = lax.broadcasted_iota(jnp.int32, x.shape, 0)
    first = pl.program_id(0) == 0
    return jnp.where(jnp.logical_and(rows == 0, first), 0.0, x)


def _tc_pre_body(fs_ref, ef_ref, wz1, wze, wr1, wre, wh1, whe, bz, bh, urT,
                 pz_ref, rm_ref, ph_ref, tab_ref):
    F = fs_ref[...]
    Ef = ef_ref[...]
    pz = _dot(F, wz1[...]) + _dot(Ef, wze[...]) + bz[...]
    rm = _dot(F, wr1[...]) + _dot(Ef, wre[...])
    ph = _dot(F, wh1[...]) + _dot(Ef, whe[...]) + bh[...]
    pz_ref[...] = pz
    rm_ref[...] = -rm
    ph_ref[...] = ph
    h1 = jax.nn.sigmoid(pz) * jnp.tanh(ph)
    h1 = _mask_row0(h1)
    tab_ref[:, :H] = h1
    tab_ref[:, H:] = _dot(h1, -urT[...])


@functools.cache
def _tc_pre():
    rspec = lambda w: pl.BlockSpec((RB, w), lambda i: (i, 0))
    wspec = pl.BlockSpec((H, H), lambda i: (0, 0))
    espec = pl.BlockSpec((EF, H), lambda i: (0, 0))
    bspec = pl.BlockSpec((1, H), lambda i: (0, 0))
    return pl.pallas_call(
        _tc_pre_body,
        grid=(E // RB,),
        in_specs=[rspec(H), rspec(EF), wspec, espec, wspec, espec, wspec,
                  espec, bspec, bspec, wspec],
        out_specs=[rspec(H), rspec(H), rspec(H), rspec(2 * H)],
        out_shape=[jax.ShapeDtypeStruct((E, H), F32)] * 3
        + [jax.ShapeDtypeStruct((E, 2 * H), F32)],
    )


def _tc_gru_body(sum_ref, pz_ref, ph_ref, wz2, wh2, urT, out_ref, *, last):
    s_h = sum_ref[:, :H]
    s_g = sum_ref[:, H:]
    z = jax.nn.sigmoid(pz_ref[...] + _dot(s_h, wz2[...]))
    p = jnp.tanh(ph_ref[...] + _dot(s_g, wh2[...]))
    h = (1.0 - z) * s_h + z * p
    h = _mask_row0(h)
    if last:
        out_ref[...] = h
    else:
        out_ref[:, :H] = h
        out_ref[:, H:] = _dot(h, -urT[...])


@functools.cache
def _tc_gru(last):
    rspec = lambda w: pl.BlockSpec((RB, w), lambda i: (i, 0))
    wspec = pl.BlockSpec((H, H), lambda i: (0, 0))
    ow = H if last else 2 * H
    specs = [rspec(2 * H), rspec(H), rspec(H), wspec, wspec, wspec]
    return pl.pallas_call(
        functools.partial(_tc_gru_body, last=last),
        grid=(E // RB,),
        in_specs=specs,
        out_specs=rspec(ow),
        out_shape=jax.ShapeDtypeStruct((E, ow), F32),
    )


def _tc_out_body(fn_ref, nei_ref, wo1, wo2, bo, hatom_ref, hmol_ref):
    x = _dot(fn_ref[0], wo1[...]) + _dot(nei_ref[0], wo2[...]) + bo[...]
    x = jnp.maximum(x, 0.0)
    x = _mask_row0(x)
    hatom_ref[0] = x
    hmol_ref[0] = jnp.sum(x, axis=0, keepdims=True)


@functools.cache
def _tc_out():
    rspec = pl.BlockSpec((1, MOLSZ, H), lambda i: (i, 0, 0))
    wspec = pl.BlockSpec((H, H), lambda i: (0, 0))
    bspec = pl.BlockSpec((1, H), lambda i: (0, 0))
    return pl.pallas_call(
        _tc_out_body,
        grid=(NMOL,),
        in_specs=[rspec, rspec, wspec, wspec, bspec],
        out_specs=[rspec, pl.BlockSpec((1, 1, H), lambda i: (i, 0, 0))],
        out_shape=[jax.ShapeDtypeStruct((NMOL, MOLSZ, H), F32),
                   jax.ShapeDtypeStruct((NMOL, 1, H), F32)],
    )


def kernel(fnode, fmess, agraph, bgraph, atom_scope, W_z, b_z, W_r, U_r,
           W_h, b_h, W_o, b_o):
    src = fmess[:, 0].astype(jnp.int32)
    efeat = fmess[:, 2:]
    bgT = bgraph.T.reshape(-1)
    agT = agraph.T.reshape(-1)

    wz1 = W_z[:, :H].T
    wze = W_z[:, H:H + EF].T
    wz2 = W_z[:, H + EF:].T
    wr1 = W_r[:, :H].T
    wre = W_r[:, H:].T
    wh1 = W_h[:, :H].T
    whe = W_h[:, H:H + EF].T
    wh2 = W_h[:, H + EF:].T
    wo1 = W_o[:, :H].T
    wo2 = W_o[:, H:].T
    urT = U_r.T
    bz = b_z.reshape(1, H)
    bh = b_h.reshape(1, H)
    bo = b_o.reshape(1, H)

    fsrc = _sc_gather_rows()(fnode, src)
    pz, rm, ph, tab = _tc_pre()(fsrc, efeat, wz1, wze, wr1, wre, wh1, whe,
                                bz, bh, urT)
    sums = _sc_msg()(tab, rm, bgT)
    tab = _tc_gru(False)(sums, pz, ph, wz2, wh2, urT)
    sums = _sc_msg()(tab, rm, bgT)
    h = _tc_gru(True)(sums, pz, ph, wz2, wh2, urT)
    nei = _sc_nbr()(h, agT)
    hatom3, hmol3 = _tc_out()(fnode.reshape(NMOL, MOLSZ, H),
                              nei.reshape(NMOL, MOLSZ, H), wo1, wo2, bo)
    return (hmol3.reshape(NMOL, H), hatom3.reshape(N, H))


# speedup vs baseline: 1.7781x; 1.7781x over previous
import functools

import jax
import jax.numpy as jnp
from jax import lax
from jax.experimental import pallas as pl
from jax.experimental.pallas import tpu as pltpu
from jax.experimental.pallas import tpu_sc as plsc

E = 160000
N = 10000
H = 128
EF = 16
NB = 6
NMOL = 100
MOLSZ = 100

NC = 2
NS = 16
NW = NC * NS
EPW = E // NW
CE = 40
NCHUNK = EPW // CE
RN = 40
NODE_CHUNKS = N // RN
RB = 800
F32 = jnp.float32


def _mesh():
    return plsc.VectorSubcoreMesh(
        core_axis_name="c", subcore_axis_name="s", num_cores=NC, num_subcores=NS
    )


def _wid():
    return lax.axis_index("s") * NC + lax.axis_index("c")


def _sigmoid16(x):
    return 1.0 / (1.0 + jnp.exp(-x))


CG = 128
NCG = EPW // CG
TAILG = EPW - NCG * CG


@functools.cache
def _sc_gather_rows():
    @functools.partial(
        pl.kernel,
        out_type=jax.ShapeDtypeStruct((E, H), F32),
        mesh=_mesh(),
        scratch_types=[
            pltpu.VMEM((EPW,), jnp.int32),
            pltpu.VMEM((3, CG, H), F32),
            pltpu.SemaphoreType.DMA,
            pltpu.SemaphoreType.DMA,
            pltpu.SemaphoreType.DMA,
            pltpu.SemaphoreType.DMA,
            pltpu.SemaphoreType.DMA,
            pltpu.SemaphoreType.DMA,
        ],
    )
    def k(table_hbm, idx_hbm, out_hbm, idx_all, rows, g0, g1, g2, o0, o1, o2):
        gsems = (g0, g1, g2)
        osems = (o0, o1, o2)
        w0 = _wid() * EPW
        pltpu.sync_copy(idx_hbm.at[pl.ds(w0, EPW)], idx_all)

        def gcp(ci, b, n):
            off = pl.multiple_of(ci * CG, 8)
            return pltpu.make_async_copy(
                table_hbm.at[idx_all.at[pl.ds(off, n)]],
                rows.at[b, pl.ds(0, n)], gsems[b])

        def ow_start(ci, b, n):
            off = pl.multiple_of(ci * CG, 8)
            pltpu.async_copy(rows.at[b, pl.ds(0, n)],
                             out_hbm.at[pl.ds(w0 + off, n)], osems[b])

        def ow_wait(b, n):
            pltpu.make_async_copy(rows.at[b, pl.ds(0, n)],
                                  out_hbm.at[pl.ds(0, n)], osems[b]).wait()

        gcp(0, 0, CG).start()

        def outer(i, carry):
            for b in range(3):
                ci = i * 3 + b
                gcp(ci, b, CG).wait()
                nb = (b + 1) % 3

                @pl.when(ci + 1 < NCG)
                def _():
                    @pl.when(ci >= 2)
                    def _():
                        ow_wait(nb, CG)

                    gcp(ci + 1, nb, CG).start()

                ow_start(ci, b, CG)
            return carry

        lax.fori_loop(0, NCG // 3, outer, 0)
        ow_wait(1, CG)
        ow_wait(2, CG)
        ow_wait(0, CG)

        gcp(NCG, 0, TAILG).start()
        gcp(NCG, 0, TAILG).wait()
        ow_start(NCG, 0, TAILG)
        ow_wait(0, TAILG)

    return k


CE2 = 24
NCH2 = EPW // CE2
TAIL2 = EPW - NCH2 * CE2


@functools.cache
def _sc_msg():
    @functools.partial(
        pl.kernel,
        out_type=jax.ShapeDtypeStruct((E, 2 * H), F32),
        mesh=_mesh(),
        scratch_types=[
            pltpu.VMEM((NB * EPW,), jnp.int32),
            pltpu.VMEM((2, NB, CE2, 2 * H), F32),
            pltpu.VMEM((2, CE2, H), F32),
            pltpu.VMEM((2, CE2, 2 * H), F32),
            pltpu.SemaphoreType.DMA,
            pltpu.SemaphoreType.DMA,
            pltpu.SemaphoreType.DMA,
            pltpu.SemaphoreType.DMA,
        ],
    )
    def k(tab_hbm, rm_hbm, bgT_hbm, out_hbm, idx_all, gb, rmv, ob,
          gs0, gs1, os0, os1):
        gsems = (gs0, gs1)
        osems = (os0, os1)
        w0 = _wid() * EPW
        for j in range(NB):
            pltpu.sync_copy(bgT_hbm.at[pl.ds(j * E + w0, EPW)],
                            idx_all.at[pl.ds(j * EPW, EPW)])

        def gather_cps(ci, b, n):
            off = pl.multiple_of(ci * CE2, 8)
            cps = [
                pltpu.make_async_copy(
                    tab_hbm.at[idx_all.at[pl.ds(j * EPW + off, n)]],
                    gb.at[b, j, pl.ds(0, n)], gsems[b])
                for j in range(NB)
            ]
            cps.append(pltpu.make_async_copy(
                rm_hbm.at[pl.ds(w0 + off, n)],
                rmv.at[b, pl.ds(0, n)], gsems[b]))
            return cps

        def owrite_start(ci, b, n):
            off = pl.multiple_of(ci * CE2, 8)
            pltpu.async_copy(ob.at[b, pl.ds(0, n)],
                             out_hbm.at[pl.ds(w0 + off, n)], osems[b])

        def owrite_wait(b, n):
            pltpu.make_async_copy(ob.at[b, pl.ds(0, n)],
                                  out_hbm.at[pl.ds(0, n)], osems[b]).wait()

        def compute(b, n):
            def edge(e, ecarry):
                for sl in range(H // 16):
                    o = sl * 16
                    rv = rmv[b, e, pl.ds(o, 16)]
                    accs = jnp.zeros((16,), F32)
                    accg = jnp.zeros((16,), F32)
                    for j in range(NB):
                        hv = gb[b, j, e, pl.ds(o, 16)]
                        uv = gb[b, j, e, pl.ds(H + o, 16)]
                        g = 1.0 / (1.0 + jnp.exp(rv + uv))
                        accs = accs + hv
                        accg = accg + g * hv
                    ob[b, e, pl.ds(o, 16)] = accs
                    ob[b, e, pl.ds(H + o, 16)] = accg
                return ecarry

            lax.fori_loop(0, n, edge, 0)

        for cp in gather_cps(0, 0, CE2):
            cp.start()

        def outer(i, carry):
            for b in range(2):
                ci = i * 2 + b
                for cp in gather_cps(ci, b, CE2):
                    cp.wait()

                @pl.when(ci + 1 < NCH2)
                def _():
                    for cp in gather_cps(ci + 1, 1 - b, CE2):
                        cp.start()

                @pl.when(ci >= 2)
                def _():
                    owrite_wait(b, CE2)

                compute(b, CE2)
                owrite_start(ci, b, CE2)
            return carry

        lax.fori_loop(0, NCH2 // 2, outer, 0)
        owrite_wait(0, CE2)
        owrite_wait(1, CE2)

        for cp in gather_cps(NCH2, 0, TAIL2):
            cp.start()
        for cp in gather_cps(NCH2, 0, TAIL2):
            cp.wait()
        compute(0, TAIL2)
        owrite_start(NCH2, 0, TAIL2)
        owrite_wait(0, TAIL2)

    return k


@functools.cache
def _sc_nbr():
    @functools.partial(
        pl.kernel,
        out_type=jax.ShapeDtypeStruct((N, H), F32),
        mesh=_mesh(),
        scratch_types=[
            pltpu.VMEM((NB, RN), jnp.int32),
            pltpu.VMEM((NB, RN, H), F32),
            pltpu.VMEM((RN, H), F32),
            pltpu.SemaphoreType.DMA,
        ],
    )
    def k(h_hbm, agT_hbm, out_hbm, idx_v, gb_v, ob_v, sem):
        w = _wid()
        steps = (NODE_CHUNKS + NW - 1) // NW

        def step(si, carry):
            ci = w + si * NW

            @pl.when(ci < NODE_CHUNKS)
            def _():
                base = pl.multiple_of(ci * RN, 8)
                for j in range(NB):
                    pltpu.sync_copy(agT_hbm.at[pl.ds(j * N + base, RN)],
                                    idx_v.at[j])
                cps = [
                    pltpu.async_copy(h_hbm.at[idx_v.at[j]], gb_v.at[j], sem)
                    for j in range(NB)
                ]
                for cp in cps:
                    cp.wait()

                def node(e, ecarry):
                    for sl in range(H // 16):
                        o = sl * 16
                        acc = jnp.zeros((16,), F32)
                        for j in range(NB):
                            acc = acc + gb_v[j, e, pl.ds(o, 16)]
                        ob_v[e, pl.ds(o, 16)] = acc
                    return ecarry

                lax.fori_loop(0, RN, node, 0)
                pltpu.sync_copy(ob_v, out_hbm.at[pl.ds(base, RN)])

            return carry

        lax.fori_loop(0, steps, step, 0)

    return k


def _dot(a, b):
    return jnp.dot(a, b, preferred_element_type=F32)


def _mask_row0(x):
    rows = lax.broadcasted_iota(jnp.int32, x.shape, 0)
    first = pl.program_id(0) == 0
    return jnp.where(jnp.logical_and(rows == 0, first), 0.0, x)


def _tc_pre_body(fs_ref, ef_ref, wz1, wze, wr1, wre, wh1, whe, bz, bh, urT,
                 pz_ref, rm_ref, ph_ref, tab_ref):
    F = fs_ref[...]
    Ef = ef_ref[...]
    pz = _dot(F, wz1[...]) + _dot(Ef, wze[...]) + bz[...]
    rm = _dot(F, wr1[...]) + _dot(Ef, wre[...])
    ph = _dot(F, wh1[...]) + _dot(Ef, whe[...]) + bh[...]
    pz_ref[...] = pz
    rm_ref[...] = -rm
    ph_ref[...] = ph
    h1 = jax.nn.sigmoid(pz) * jnp.tanh(ph)
    h1 = _mask_row0(h1)
    tab_ref[:, :H] = h1
    tab_ref[:, H:] = _dot(h1, -urT[...])


@functools.cache
def _tc_pre():
    rspec = lambda w: pl.BlockSpec((RB, w), lambda i: (i, 0))
    wspec = pl.BlockSpec((H, H), lambda i: (0, 0))
    espec = pl.BlockSpec((EF, H), lambda i: (0, 0))
    bspec = pl.BlockSpec((1, H), lambda i: (0, 0))
    return pl.pallas_call(
        _tc_pre_body,
        grid=(E // RB,),
        in_specs=[rspec(H), rspec(EF), wspec, espec, wspec, espec, wspec,
                  espec, bspec, bspec, wspec],
        out_specs=[rspec(H), rspec(H), rspec(H), rspec(2 * H)],
        out_shape=[jax.ShapeDtypeStruct((E, H), F32)] * 3
        + [jax.ShapeDtypeStruct((E, 2 * H), F32)],
    )


def _tc_gru_body(sum_ref, pz_ref, ph_ref, wz2, wh2, urT, out_ref, *, last):
    s_h = sum_ref[:, :H]
    s_g = sum_ref[:, H:]
    z = jax.nn.sigmoid(pz_ref[...] + _dot(s_h, wz2[...]))
    p = jnp.tanh(ph_ref[...] + _dot(s_g, wh2[...]))
    h = (1.0 - z) * s_h + z * p
    h = _mask_row0(h)
    if last:
        out_ref[...] = h
    else:
        out_ref[:, :H] = h
        out_ref[:, H:] = _dot(h, -urT[...])


@functools.cache
def _tc_gru(last):
    rspec = lambda w: pl.BlockSpec((RB, w), lambda i: (i, 0))
    wspec = pl.BlockSpec((H, H), lambda i: (0, 0))
    ow = H if last else 2 * H
    specs = [rspec(2 * H), rspec(H), rspec(H), wspec, wspec, wspec]
    return pl.pallas_call(
        functools.partial(_tc_gru_body, last=last),
        grid=(E // RB,),
        in_specs=specs,
        out_specs=rspec(ow),
        out_shape=jax.ShapeDtypeStruct((E, ow), F32),
    )


def _tc_out_body(fn_ref, nei_ref, wo1, wo2, bo, hatom_ref, hmol_ref):
    x = _dot(fn_ref[0], wo1[...]) + _dot(nei_ref[0], wo2[...]) + bo[...]
    x = jnp.maximum(x, 0.0)
    x = _mask_row0(x)
    hatom_ref[0] = x
    hmol_ref[0] = jnp.sum(x, axis=0, keepdims=True)


@functools.cache
def _tc_out():
    rspec = pl.BlockSpec((1, MOLSZ, H), lambda i: (i, 0, 0))
    wspec = pl.BlockSpec((H, H), lambda i: (0, 0))
    bspec = pl.BlockSpec((1, H), lambda i: (0, 0))
    return pl.pallas_call(
        _tc_out_body,
        grid=(NMOL,),
        in_specs=[rspec, rspec, wspec, wspec, bspec],
        out_specs=[rspec, pl.BlockSpec((1, 1, H), lambda i: (i, 0, 0))],
        out_shape=[jax.ShapeDtypeStruct((NMOL, MOLSZ, H), F32),
                   jax.ShapeDtypeStruct((NMOL, 1, H), F32)],
    )


def kernel(fnode, fmess, agraph, bgraph, atom_scope, W_z, b_z, W_r, U_r,
           W_h, b_h, W_o, b_o):
    src = fmess[:, 0].astype(jnp.int32)
    efeat = fmess[:, 2:]
    bgT = bgraph.T.reshape(-1)
    agT = agraph.T.reshape(-1)

    wz1 = W_z[:, :H].T
    wze = W_z[:, H:H + EF].T
    wz2 = W_z[:, H + EF:].T
    wr1 = W_r[:, :H].T
    wre = W_r[:, H:].T
    wh1 = W_h[:, :H].T
    whe = W_h[:, H:H + EF].T
    wh2 = W_h[:, H + EF:].T
    wo1 = W_o[:, :H].T
    wo2 = W_o[:, H:].T
    urT = U_r.T
    bz = b_z.reshape(1, H)
    bh = b_h.reshape(1, H)
    bo = b_o.reshape(1, H)

    fsrc = _sc_gather_rows()(fnode, src)
    pz, rm, ph, tab = _tc_pre()(fsrc, efeat, wz1, wze, wr1, wre, wh1, whe,
                                bz, bh, urT)
    sums = _sc_msg()(tab, rm, bgT)
    tab = _tc_gru(False)(sums, pz, ph, wz2, wh2, urT)
    sums = _sc_msg()(tab, rm, bgT)
    h = _tc_gru(True)(sums, pz, ph, wz2, wh2, urT)
    nei = _sc_nbr()(h, agT)
    hatom3, hmol3 = _tc_out()(fnode.reshape(NMOL, MOLSZ, H),
                              nei.reshape(NMOL, MOLSZ, H), wo1, wo2, bo)
    return (hmol3.reshape(NMOL, H), hatom3.reshape(N, H))


# speedup vs baseline: 2.6841x; 1.5096x over previous
import functools

import jax
import jax.numpy as jnp
from jax import lax
from jax.experimental import pallas as pl
from jax.experimental.pallas import tpu as pltpu
from jax.experimental.pallas import tpu_sc as plsc

E = 160000
N = 10000
H = 128
EF = 16
NB = 6
NMOL = 100
MOLSZ = 100

NC = 2
NS = 16
NW = NC * NS
EPW = E // NW
CE = 40
NCHUNK = EPW // CE
RN = 40
NODE_CHUNKS = N // RN
RB = 800
F32 = jnp.float32


def _mesh():
    return plsc.VectorSubcoreMesh(
        core_axis_name="c", subcore_axis_name="s", num_cores=NC, num_subcores=NS
    )


def _wid():
    return lax.axis_index("s") * NC + lax.axis_index("c")


def _sigmoid16(x):
    return 1.0 / (1.0 + jnp.exp(-x))


CG = 128
NCG = EPW // CG
TAILG = EPW - NCG * CG


@functools.cache
def _sc_gather_rows():
    @functools.partial(
        pl.kernel,
        out_type=jax.ShapeDtypeStruct((E, H), F32),
        mesh=_mesh(),
        scratch_types=[
            pltpu.VMEM((EPW,), jnp.int32),
            pltpu.VMEM((3, CG, H), F32),
            pltpu.SemaphoreType.DMA,
            pltpu.SemaphoreType.DMA,
            pltpu.SemaphoreType.DMA,
            pltpu.SemaphoreType.DMA,
            pltpu.SemaphoreType.DMA,
            pltpu.SemaphoreType.DMA,
        ],
    )
    def k(table_hbm, idx_hbm, out_hbm, idx_all, rows, g0, g1, g2, o0, o1, o2):
        gsems = (g0, g1, g2)
        osems = (o0, o1, o2)
        w0 = _wid() * EPW
        pltpu.sync_copy(idx_hbm.at[pl.ds(w0, EPW)], idx_all)

        def gcp(ci, b, n):
            off = pl.multiple_of(ci * CG, 8)
            return pltpu.make_async_copy(
                table_hbm.at[idx_all.at[pl.ds(off, n)]],
                rows.at[b, pl.ds(0, n)], gsems[b])

        def ow_start(ci, b, n):
            off = pl.multiple_of(ci * CG, 8)
            pltpu.async_copy(rows.at[b, pl.ds(0, n)],
                             out_hbm.at[pl.ds(w0 + off, n)], osems[b])

        def ow_wait(b, n):
            pltpu.make_async_copy(rows.at[b, pl.ds(0, n)],
                                  out_hbm.at[pl.ds(0, n)], osems[b]).wait()

        gcp(0, 0, CG).start()

        def outer(i, carry):
            for b in range(3):
                ci = i * 3 + b
                gcp(ci, b, CG).wait()
                nb = (b + 1) % 3

                @pl.when(ci + 1 < NCG)
                def _():
                    @pl.when(ci >= 2)
                    def _():
                        ow_wait(nb, CG)

                    gcp(ci + 1, nb, CG).start()

                ow_start(ci, b, CG)
            return carry

        lax.fori_loop(0, NCG // 3, outer, 0)
        ow_wait(1, CG)
        ow_wait(2, CG)
        ow_wait(0, CG)

        gcp(NCG, 0, TAILG).start()
        gcp(NCG, 0, TAILG).wait()
        ow_start(NCG, 0, TAILG)
        ow_wait(0, TAILG)

    return k


CE2 = 24
NCH2 = EPW // CE2
TAIL2 = EPW - NCH2 * CE2


@functools.cache
def _sc_msg():
    @functools.partial(
        pl.kernel,
        out_type=jax.ShapeDtypeStruct((E, 2 * H), F32),
        mesh=_mesh(),
        scratch_types=[
            pltpu.VMEM((NB * EPW,), jnp.int32),
            pltpu.VMEM((2, NB, CE2, 2 * H), F32),
            pltpu.VMEM((2, CE2, H), F32),
            pltpu.VMEM((2, CE2, 2 * H), F32),
            pltpu.SemaphoreType.DMA,
            pltpu.SemaphoreType.DMA,
            pltpu.SemaphoreType.DMA,
            pltpu.SemaphoreType.DMA,
        ],
    )
    def k(tab_hbm, rm_hbm, bgT_hbm, out_hbm, idx_all, gb, rmv, ob,
          gs0, gs1, os0, os1):
        gsems = (gs0, gs1)
        osems = (os0, os1)
        w0 = _wid() * EPW
        for j in range(NB):
            pltpu.sync_copy(bgT_hbm.at[pl.ds(j * E + w0, EPW)],
                            idx_all.at[pl.ds(j * EPW, EPW)])

        def gather_cps(ci, b, n):
            off = pl.multiple_of(ci * CE2, 8)
            cps = [
                pltpu.make_async_copy(
                    tab_hbm.at[idx_all.at[pl.ds(j * EPW + off, n)]],
                    gb.at[b, j, pl.ds(0, n)], gsems[b])
                for j in range(NB)
            ]
            cps.append(pltpu.make_async_copy(
                rm_hbm.at[pl.ds(w0 + off, n)],
                rmv.at[b, pl.ds(0, n)], gsems[b]))
            return cps

        def owrite_start(ci, b, n):
            off = pl.multiple_of(ci * CE2, 8)
            pltpu.async_copy(ob.at[b, pl.ds(0, n)],
                             out_hbm.at[pl.ds(w0 + off, n)], osems[b])

        def owrite_wait(b, n):
            pltpu.make_async_copy(ob.at[b, pl.ds(0, n)],
                                  out_hbm.at[pl.ds(0, n)], osems[b]).wait()

        def compute(b, n):
            @plsc.parallel_loop(0, n)
            def edge(e):
                for sl in range(H // 16):
                    o = sl * 16
                    rv = rmv[b, e, pl.ds(o, 16)]
                    accs = jnp.zeros((16,), F32)
                    accg = jnp.zeros((16,), F32)
                    for j in range(NB):
                        hv = gb[b, j, e, pl.ds(o, 16)]
                        uv = gb[b, j, e, pl.ds(H + o, 16)]
                        g = 1.0 / (1.0 + jnp.exp(rv + uv))
                        accs = accs + hv
                        accg = accg + g * hv
                    ob[b, e, pl.ds(o, 16)] = accs
                    ob[b, e, pl.ds(H + o, 16)] = accg

        for cp in gather_cps(0, 0, CE2):
            cp.start()

        def outer(i, carry):
            for b in range(2):
                ci = i * 2 + b
                for cp in gather_cps(ci, b, CE2):
                    cp.wait()

                @pl.when(ci + 1 < NCH2)
                def _():
                    for cp in gather_cps(ci + 1, 1 - b, CE2):
                        cp.start()

                @pl.when(ci >= 2)
                def _():
                    owrite_wait(b, CE2)

                compute(b, CE2)
                owrite_start(ci, b, CE2)
            return carry

        lax.fori_loop(0, NCH2 // 2, outer, 0)
        owrite_wait(0, CE2)
        owrite_wait(1, CE2)

        for cp in gather_cps(NCH2, 0, TAIL2):
            cp.start()
        for cp in gather_cps(NCH2, 0, TAIL2):
            cp.wait()
        compute(0, TAIL2)
        owrite_start(NCH2, 0, TAIL2)
        owrite_wait(0, TAIL2)

    return k


@functools.cache
def _sc_nbr():
    @functools.partial(
        pl.kernel,
        out_type=jax.ShapeDtypeStruct((N, H), F32),
        mesh=_mesh(),
        scratch_types=[
            pltpu.VMEM((NB, RN), jnp.int32),
            pltpu.VMEM((NB, RN, H), F32),
            pltpu.VMEM((RN, H), F32),
            pltpu.SemaphoreType.DMA,
        ],
    )
    def k(h_hbm, agT_hbm, out_hbm, idx_v, gb_v, ob_v, sem):
        w = _wid()
        steps = (NODE_CHUNKS + NW - 1) // NW

        def step(si, carry):
            ci = w + si * NW

            @pl.when(ci < NODE_CHUNKS)
            def _():
                base = pl.multiple_of(ci * RN, 8)
                for j in range(NB):
                    pltpu.sync_copy(agT_hbm.at[pl.ds(j * N + base, RN)],
                                    idx_v.at[j])
                cps = [
                    pltpu.async_copy(h_hbm.at[idx_v.at[j]], gb_v.at[j], sem)
                    for j in range(NB)
                ]
                for cp in cps:
                    cp.wait()

                def node(e, ecarry):
                    for sl in range(H // 16):
                        o = sl * 16
                        acc = jnp.zeros((16,), F32)
                        for j in range(NB):
                            acc = acc + gb_v[j, e, pl.ds(o, 16)]
                        ob_v[e, pl.ds(o, 16)] = acc
                    return ecarry

                lax.fori_loop(0, RN, node, 0)
                pltpu.sync_copy(ob_v, out_hbm.at[pl.ds(base, RN)])

            return carry

        lax.fori_loop(0, steps, step, 0)

    return k


def _dot(a, b):
    return jnp.dot(a, b, preferred_element_type=F32)


def _mask_row0(x):
    rows = lax.broadcasted_iota(jnp.int32, x.shape, 0)
    first = pl.program_id(0) == 0
    return jnp.where(jnp.logical_and(rows == 0, first), 0.0, x)


def _tc_pre_body(fs_ref, ef_ref, wz1, wze, wr1, wre, wh1, whe, bz, bh, urT,
                 pz_ref, rm_ref, ph_ref, tab_ref):
    F = fs_ref[...]
    Ef = ef_ref[...]
    pz = _dot(F, wz1[...]) + _dot(Ef, wze[...]) + bz[...]
    rm = _dot(F, wr1[...]) + _dot(Ef, wre[...])
    ph = _dot(F, wh1[...]) + _dot(Ef, whe[...]) + bh[...]
    pz_ref[...] = pz
    rm_ref[...] = -rm
    ph_ref[...] = ph
    h1 = jax.nn.sigmoid(pz) * jnp.tanh(ph)
    h1 = _mask_row0(h1)
    tab_ref[:, :H] = h1
    tab_ref[:, H:] = _dot(h1, -urT[...])


@functools.cache
def _tc_pre():
    rspec = lambda w: pl.BlockSpec((RB, w), lambda i: (i, 0))
    wspec = pl.BlockSpec((H, H), lambda i: (0, 0))
    espec = pl.BlockSpec((EF, H), lambda i: (0, 0))
    bspec = pl.BlockSpec((1, H), lambda i: (0, 0))
    return pl.pallas_call(
        _tc_pre_body,
        grid=(E // RB,),
        in_specs=[rspec(H), rspec(EF), wspec, espec, wspec, espec, wspec,
                  espec, bspec, bspec, wspec],
        out_specs=[rspec(H), rspec(H), rspec(H), rspec(2 * H)],
        out_shape=[jax.ShapeDtypeStruct((E, H), F32)] * 3
        + [jax.ShapeDtypeStruct((E, 2 * H), F32)],
    )


def _tc_gru_body(sum_ref, pz_ref, ph_ref, wz2, wh2, urT, out_ref, *, last):
    s_h = sum_ref[:, :H]
    s_g = sum_ref[:, H:]
    z = jax.nn.sigmoid(pz_ref[...] + _dot(s_h, wz2[...]))
    p = jnp.tanh(ph_ref[...] + _dot(s_g, wh2[...]))
    h = (1.0 - z) * s_h + z * p
    h = _mask_row0(h)
    if last:
        out_ref[...] = h
    else:
        out_ref[:, :H] = h
        out_ref[:, H:] = _dot(h, -urT[...])


@functools.cache
def _tc_gru(last):
    rspec = lambda w: pl.BlockSpec((RB, w), lambda i: (i, 0))
    wspec = pl.BlockSpec((H, H), lambda i: (0, 0))
    ow = H if last else 2 * H
    specs = [rspec(2 * H), rspec(H), rspec(H), wspec, wspec, wspec]
    return pl.pallas_call(
        functools.partial(_tc_gru_body, last=last),
        grid=(E // RB,),
        in_specs=specs,
        out_specs=rspec(ow),
        out_shape=jax.ShapeDtypeStruct((E, ow), F32),
    )


def _tc_out_body(fn_ref, nei_ref, wo1, wo2, bo, hatom_ref, hmol_ref):
    x = _dot(fn_ref[0], wo1[...]) + _dot(nei_ref[0], wo2[...]) + bo[...]
    x = jnp.maximum(x, 0.0)
    x = _mask_row0(x)
    hatom_ref[0] = x
    hmol_ref[0] = jnp.sum(x, axis=0, keepdims=True)


@functools.cache
def _tc_out():
    rspec = pl.BlockSpec((1, MOLSZ, H), lambda i: (i, 0, 0))
    wspec = pl.BlockSpec((H, H), lambda i: (0, 0))
    bspec = pl.BlockSpec((1, H), lambda i: (0, 0))
    return pl.pallas_call(
        _tc_out_body,
        grid=(NMOL,),
        in_specs=[rspec, rspec, wspec, wspec, bspec],
        out_specs=[rspec, pl.BlockSpec((1, 1, H), lambda i: (i, 0, 0))],
        out_shape=[jax.ShapeDtypeStruct((NMOL, MOLSZ, H), F32),
                   jax.ShapeDtypeStruct((NMOL, 1, H), F32)],
    )


def kernel(fnode, fmess, agraph, bgraph, atom_scope, W_z, b_z, W_r, U_r,
           W_h, b_h, W_o, b_o):
    src = fmess[:, 0].astype(jnp.int32)
    efeat = fmess[:, 2:]
    bgT = bgraph.T.reshape(-1)
    agT = agraph.T.reshape(-1)

    wz1 = W_z[:, :H].T
    wze = W_z[:, H:H + EF].T
    wz2 = W_z[:, H + EF:].T
    wr1 = W_r[:, :H].T
    wre = W_r[:, H:].T
    wh1 = W_h[:, :H].T
    whe = W_h[:, H:H + EF].T
    wh2 = W_h[:, H + EF:].T
    wo1 = W_o[:, :H].T
    wo2 = W_o[:, H:].T
    urT = U_r.T
    bz = b_z.reshape(1, H)
    bh = b_h.reshape(1, H)
    bo = b_o.reshape(1, H)

    fsrc = _sc_gather_rows()(fnode, src)
    pz, rm, ph, tab = _tc_pre()(fsrc, efeat, wz1, wze, wr1, wre, wh1, whe,
                                bz, bh, urT)
    sums = _sc_msg()(tab, rm, bgT)
    tab = _tc_gru(False)(sums, pz, ph, wz2, wh2, urT)
    sums = _sc_msg()(tab, rm, bgT)
    h = _tc_gru(True)(sums, pz, ph, wz2, wh2, urT)
    nei = _sc_nbr()(h, agT)
    hatom3, hmol3 = _tc_out()(fnode.reshape(NMOL, MOLSZ, H),
                              nei.reshape(NMOL, MOLSZ, H), wo1, wo2, bo)
    return (hmol3.reshape(NMOL, H), hatom3.reshape(N, H))
